# Initial kernel scaffold; baseline (speedup 1.0000x reference)
import jax
import jax.numpy as jnp
from jax.experimental import pallas as pl


def kernel(token_observations):
    raise NotImplementedError("write your pallas kernel here")


# speedup vs baseline: 4.8554x; 4.8554x over previous
import jax
import jax.numpy as jnp
from jax import lax
from jax.experimental import pallas as pl
from jax.experimental.pallas import tpu as pltpu
from jax.experimental.pallas import tpu_sc as plsc

_NUM_LAYERS = 64
_OUT_W = 16
_OUT_H = 16
_BOX = _NUM_LAYERS * _OUT_W * _OUT_H
_T = 200
_G = (_T + 15) // 16
_NW = 32

_MESH = dict(core_axis_name="c", subcore_axis_name="s", num_cores=2,
             num_subcores=16)


def _decode_body(tok_hbm, keys_hbm, vals_hbm, tok_v, keys_v, vals_v, sem):
    wid = lax.axis_index("s") * 2 + lax.axis_index("c")
    rows = tok_hbm.shape[0] // _NW
    base_row = wid * rows
    lanes = lax.iota(jnp.int32, 16)

    def _row(r, _):
        row = base_row + r
        pltpu.sync_copy(tok_hbm.at[row], tok_v)
        for g in range(_G):
            pos = lanes + (g * 16)
            pidx = jnp.minimum(pos, _T - 1) * 3
            obs0 = plsc.load_gather(tok_v, [pidx])
            atr = plsc.load_gather(tok_v, [pidx + 1])
            val = plsc.load_gather(tok_v, [pidx + 2])
            coords = obs0 & 255
            valid = (coords != 255) & (atr < _NUM_LAYERS)
            sidx = jnp.where(valid, atr * 256 + coords, 0)
            sval = jnp.where(valid, val.astype(jnp.float32), 0.0)
            keys_v[pl.ds(g * 16, 16)] = sidx + row * _BOX
            vals_v[pl.ds(g * 16, 16)] = sval
        pltpu.sync_copy(keys_v.at[pl.ds(0, _T)], keys_hbm.at[pl.ds(row * _T, _T)])
        pltpu.sync_copy(vals_v.at[pl.ds(0, _T)], vals_hbm.at[pl.ds(row * _T, _T)])
        return 0

    lax.fori_loop(0, rows, _row, 0)


def _scatter_body(sk_hbm, sv_hbm, out_hbm, keys_v, vals_v, box_v, sem):
    wid = lax.axis_index("s") * 2 + lax.axis_index("c")
    b_tt = out_hbm.shape[0]
    rows = b_tt // _NW
    base_row = wid * rows
    zero16 = jnp.zeros((16,), jnp.float32)
    neg1 = jnp.full((16,), -1, jnp.int32)

    def _clear(j, _):
        box_v[pl.ds(j * 16, 16)] = zero16
        return 0
    lax.fori_loop(0, _BOX // 16, _clear, 0)
    keys_v[pl.ds(_T, 16)] = neg1

    def _row(r, _):
        row = base_row + r
        pltpu.sync_copy(sk_hbm.at[pl.ds(row * _T, _T)], keys_v.at[pl.ds(0, _T)])
        pltpu.sync_copy(sv_hbm.at[pl.ds(row * _T, _T)], vals_v.at[pl.ds(0, _T)])

        for g in range(_G):
            k0 = keys_v[pl.ds(g * 16, 16)]
            k1 = keys_v[pl.ds(g * 16 + 1, 16)]
            v0 = vals_v[pl.ds(g * 16, 16)]
            m = (k0 != k1) & (k0 > -1)
            plsc.store_scatter(box_v, [k0 & (_BOX - 1)], v0, mask=m)

        pltpu.sync_copy(box_v, out_hbm.at[row])

        for g in range(_G):
            k0 = keys_v[pl.ds(g * 16, 16)]
            plsc.store_scatter(box_v, [k0 & (_BOX - 1)], zero16)
        return 0

    lax.fori_loop(0, rows, _row, 0)


def kernel(token_observations):
    b_tt = token_observations.shape[0]
    tok_flat = token_observations.reshape(b_tt, _T * 3)

    decode = pl.kernel(
        _decode_body,
        out_type=(
            jax.ShapeDtypeStruct((b_tt * _T,), jnp.int32),
            jax.ShapeDtypeStruct((b_tt * _T,), jnp.float32),
        ),
        mesh=plsc.VectorSubcoreMesh(**_MESH),
        scratch_types=[
            pltpu.VMEM((_T * 3,), jnp.int32),
            pltpu.VMEM((_G * 16,), jnp.int32),
            pltpu.VMEM((_G * 16,), jnp.float32),
            pltpu.SemaphoreType.DMA,
        ],
        compiler_params=pltpu.CompilerParams(needs_layout_passes=False),
    )
    keys1d, vals1d = decode(tok_flat)

    sk, sv = lax.sort(
        (keys1d, vals1d), dimension=0, num_keys=1, is_stable=False,
    )

    scatter = pl.kernel(
        _scatter_body,
        out_type=jax.ShapeDtypeStruct((b_tt, _BOX), jnp.float32),
        mesh=plsc.VectorSubcoreMesh(**_MESH),
        scratch_types=[
            pltpu.VMEM((_T + 16,), jnp.int32),
            pltpu.VMEM((_G * 16,), jnp.float32),
            pltpu.VMEM((_BOX,), jnp.float32),
            pltpu.SemaphoreType.DMA,
        ],
        compiler_params=pltpu.CompilerParams(needs_layout_passes=False),
    )
    out = scatter(sk, sv)
    return out.reshape(b_tt, _NUM_LAYERS, _OUT_W, _OUT_H)
